# baseline (speedup 1.0000x reference)
import functools

import jax
import jax.numpy as jnp
from jax import lax
from jax.experimental import pallas as pl
from jax.experimental.pallas import tpu as pltpu
from jax.experimental.pallas import tpu_sc as plsc

N = 1048576
NCLS = 21
NCORES = 2
NSUB = 16
LANES = 16
NW = NCORES * NSUB
ROWS_PER_W = N // NW
CHUNK = 4096
NCHUNK = ROWS_PER_W // CHUNK
GROUPS = CHUNK // LANES
UNROLL = 8
BLK = 2048


def _tc_select_body(r_ref, lab_ref, out_ref):
    rv = r_ref[...]
    labv = lab_ref[...]
    rvt = rv.T
    cls = lax.broadcasted_iota(jnp.int32, (NCLS, BLK), 0)
    mask = cls == labv[None, :]
    g = jnp.sum(jnp.where(mask, rvt, 0.0), axis=0)
    out_ref[...] = -jnp.log(g)


def _tc_select(realinput, reallabel):
    return pl.pallas_call(
        _tc_select_body,
        grid=(N // BLK,),
        in_specs=[
            pl.BlockSpec((BLK, NCLS), lambda i: (i, 0)),
            pl.BlockSpec((BLK,), lambda i: (i,)),
        ],
        out_specs=pl.BlockSpec((BLK,), lambda i: (i,)),
        out_shape=jax.ShapeDtypeStruct((N,), jnp.float32),
    )(realinput, reallabel)


def _sc_body(ll_hbm, lab_hbm, sums_out, cnts_out,
             vbuf, lbuf, sums_t, cnts_t, sv, sl):
    wid = lax.axis_index("s") * NCORES + lax.axis_index("c")
    row0 = wid * ROWS_PER_W

    z16 = jnp.zeros((LANES,), jnp.float32)
    for c in range(NCLS):
        sums_t[c, :] = z16
        cnts_t[c, :] = z16

    lane = lax.iota(jnp.int32, LANES)
    ones = jnp.ones((LANES,), jnp.float32)

    def chunk_body(k, _):
        base = row0 + k * CHUNK
        pltpu.async_copy(ll_hbm.at[pl.ds(base, CHUNK)], vbuf, sv).wait()
        pltpu.async_copy(lab_hbm.at[pl.ds(base, CHUNK)], lbuf, sl).wait()

        @plsc.parallel_loop(0, GROUPS, unroll=UNROLL)
        def _group(g):
            off = g * LANES
            labv = lbuf[pl.ds(off, LANES)]
            vals = vbuf[pl.ds(off, LANES)]
            plsc.addupdate_scatter(sums_t, [labv, lane], vals)
            plsc.addupdate_scatter(cnts_t, [labv, lane], ones)

        return 0

    lax.fori_loop(0, NCHUNK, chunk_body, 0)

    pltpu.sync_copy(sums_t, sums_out.at[wid])
    pltpu.sync_copy(cnts_t, cnts_out.at[wid])


_sc_kernel = functools.partial(
    pl.kernel,
    out_type=(
        jax.ShapeDtypeStruct((NW, NCLS, LANES), jnp.float32),
        jax.ShapeDtypeStruct((NW, NCLS, LANES), jnp.float32),
    ),
    mesh=plsc.VectorSubcoreMesh(
        core_axis_name="c", subcore_axis_name="s",
        num_cores=NCORES, num_subcores=NSUB),
    compiler_params=pltpu.CompilerParams(needs_layout_passes=False),
    scratch_types=(
        pltpu.VMEM((CHUNK,), jnp.float32),
        pltpu.VMEM((CHUNK,), jnp.int32),
        pltpu.VMEM((NCLS, LANES), jnp.float32),
        pltpu.VMEM((NCLS, LANES), jnp.float32),
        pltpu.SemaphoreType.DMA,
        pltpu.SemaphoreType.DMA,
    ),
)(_sc_body)


def _combine_body(sums_ref, cnts_ref, wl_ref, ls_ref, out_ref):
    s = jnp.sum(sums_ref[...], axis=0)
    c = jnp.sum(cnts_ref[...], axis=0)
    per_class = jnp.sum(s, axis=1, keepdims=True)
    counts = jnp.sum(c, axis=1, keepdims=True)
    present = (counts > 0.0).astype(jnp.float32)
    contrib = wl_ref[...] * (per_class[1:] + 1.0) * present[1:]
    out_ref[...] = jnp.reshape(jnp.sum(contrib) / jnp.sum(ls_ref[...]), (1, 1))


def kernel(realinput, reallabel, Wl, label_sum):
    loglab = _tc_select(realinput, reallabel)
    sums, cnts = _sc_kernel(loglab, reallabel)
    out = pl.pallas_call(
        _combine_body,
        out_shape=jax.ShapeDtypeStruct((1, 1), jnp.float32),
    )(sums, cnts, Wl.reshape(NCLS - 1, 1), label_sum.reshape(NCLS - 1, 1))
    return out[0, 0]
